# baseline (speedup 1.0000x reference)
import functools

import jax
import jax.numpy as jnp
from jax.experimental import pallas as pl
from jax.experimental.pallas import tpu as pltpu

_N_BOOKS = 16
_K = 256
_D = 32


def _vq_block_kernel(x_ref, cbm2_ref, cbhl_ref, c2t_ref, x2_ref,
                     codes_ref, recon_ref):
    x = x_ref[...]
    bt = x.shape[0]
    iota0 = jax.lax.broadcasted_iota(jnp.int32, (_K, bt), 0)
    x2t = x2_ref[...].T
    code_rows = []
    recon_cols = []
    for n in range(_N_BOOKS):
        xn = x[:, n * _D:(n + 1) * _D]
        cross_t = jax.lax.dot_general(
            cbm2_ref[n], xn, (((1,), (1,)), ((), ())),
            preferred_element_type=jnp.float32)
        dist2 = (x2t[n:n + 1, :] + c2t_ref[:, n:n + 1]) + cross_t
        score = jnp.sqrt(jnp.maximum(dist2, 0.0))
        minval = jnp.min(score, axis=0, keepdims=True)
        idx = jnp.min(jnp.where(score == minval, iota0, _K), axis=0,
                      keepdims=True)
        onehot = (iota0 == idx).astype(jnp.bfloat16)
        rec2 = jax.lax.dot_general(
            onehot, cbhl_ref[n], (((0,), (0,)), ((), ())),
            preferred_element_type=jnp.float32)
        code_rows.append(idx)
        recon_cols.append(rec2[:, :_D] + rec2[:, _D:])
    codes_t = jnp.concatenate(code_rows, axis=0)
    codes_ref[...] = codes_t.T.astype(jnp.uint8)
    recon_ref[...] = jnp.concatenate(recon_cols, axis=1)


@jax.jit
def _vq(x, codebooks):
    b, e = x.shape
    block_b = 2048
    cbm2 = -2.0 * codebooks
    cb_hi = codebooks.astype(jnp.bfloat16).astype(jnp.float32)
    cbhl = jnp.concatenate([cb_hi, codebooks - cb_hi], axis=-1)
    c2t = jax.lax.optimization_barrier(
        jnp.sum(codebooks * codebooks, axis=-1)).T
    xr = x.reshape(b, _N_BOOKS, _D)
    x2 = jax.lax.optimization_barrier(jnp.sum(xr * xr, axis=-1))
    return pl.pallas_call(
        _vq_block_kernel,
        grid=(b // block_b,),
        in_specs=[
            pl.BlockSpec((block_b, e), lambda i: (i, 0)),
            pl.BlockSpec((_N_BOOKS, _K, _D), lambda i: (0, 0, 0)),
            pl.BlockSpec((_N_BOOKS, _K, 2 * _D), lambda i: (0, 0, 0)),
            pl.BlockSpec((_K, _N_BOOKS), lambda i: (0, 0)),
            pl.BlockSpec((block_b, _N_BOOKS), lambda i: (i, 0)),
        ],
        out_specs=[
            pl.BlockSpec((block_b, _N_BOOKS), lambda i: (i, 0)),
            pl.BlockSpec((block_b, e), lambda i: (i, 0)),
        ],
        out_shape=[
            jax.ShapeDtypeStruct((b, _N_BOOKS), jnp.uint8),
            jax.ShapeDtypeStruct((b, e), jnp.float32),
        ],
    )(x, cbm2, cbhl, c2t, x2)


def kernel(x, codebooks):
    return _vq(x, codebooks)


# speedup vs baseline: 1.4582x; 1.4582x over previous
import functools

import jax
import jax.numpy as jnp
from jax.experimental import pallas as pl
from jax.experimental.pallas import tpu as pltpu

_N_BOOKS = 16
_K = 256
_D = 32


def _vq_block_kernel(x_ref, cbm2_ref, cbhl_ref, c2t_ref, x2_ref,
                     codes_ref, recon_ref):
    x = x_ref[...]
    bt = x.shape[0]
    iota0 = jax.lax.broadcasted_iota(jnp.int32, (_K, bt), 0)
    x2t = x2_ref[...].T
    code_rows = []
    recon_cols = []
    for n in range(_N_BOOKS):
        xn = x[:, n * _D:(n + 1) * _D]
        cross_t = jax.lax.dot_general(
            cbm2_ref[n], xn, (((1,), (1,)), ((), ())),
            preferred_element_type=jnp.float32)
        score = (x2t[n:n + 1, :] + c2t_ref[:, n:n + 1]) + cross_t
        minval = jnp.min(score, axis=0, keepdims=True)
        idx = jnp.min(jnp.where(score == minval, iota0, _K), axis=0,
                      keepdims=True)
        onehot = (iota0 == idx).astype(jnp.bfloat16)
        rec2 = jax.lax.dot_general(
            onehot, cbhl_ref[n], (((0,), (0,)), ((), ())),
            preferred_element_type=jnp.float32)
        code_rows.append(idx)
        recon_cols.append(rec2[:, :_D] + rec2[:, _D:])
    codes_t = jnp.concatenate(code_rows, axis=0)
    codes_ref[...] = codes_t.T.astype(jnp.uint8)
    recon_ref[...] = jnp.concatenate(recon_cols, axis=1)


@jax.jit
def _vq(x, codebooks):
    b, e = x.shape
    block_b = 2048
    cbm2 = -2.0 * codebooks
    cb_hi = codebooks.astype(jnp.bfloat16).astype(jnp.float32)
    cbhl = jnp.concatenate([cb_hi, codebooks - cb_hi], axis=-1)
    c2t = jax.lax.optimization_barrier(
        jnp.sum(codebooks * codebooks, axis=-1)).T
    xr = x.reshape(b, _N_BOOKS, _D)
    x2 = jax.lax.optimization_barrier(jnp.sum(xr * xr, axis=-1))
    return pl.pallas_call(
        _vq_block_kernel,
        grid=(b // block_b,),
        in_specs=[
            pl.BlockSpec((block_b, e), lambda i: (i, 0)),
            pl.BlockSpec((_N_BOOKS, _K, _D), lambda i: (0, 0, 0)),
            pl.BlockSpec((_N_BOOKS, _K, 2 * _D), lambda i: (0, 0, 0)),
            pl.BlockSpec((_K, _N_BOOKS), lambda i: (0, 0)),
            pl.BlockSpec((block_b, _N_BOOKS), lambda i: (i, 0)),
        ],
        out_specs=[
            pl.BlockSpec((block_b, _N_BOOKS), lambda i: (i, 0)),
            pl.BlockSpec((block_b, e), lambda i: (i, 0)),
        ],
        out_shape=[
            jax.ShapeDtypeStruct((b, _N_BOOKS), jnp.uint8),
            jax.ShapeDtypeStruct((b, e), jnp.float32),
        ],
    )(x, cbm2, cbhl, c2t, x2)


def kernel(x, codebooks):
    return _vq(x, codebooks)
